# baseline (speedup 1.0000x reference)
import jax
import jax.numpy as jnp
from jax.experimental import pallas as pl

_BT = 1024


def _gate_kernel(x_ref, w_ref, o_ref):
    o_ref[...] = jax.lax.dot_general(
        x_ref[...].astype(jnp.bfloat16), w_ref[...].astype(jnp.bfloat16),
        dimension_numbers=(((1,), (1,)), ((), ())),
        preferred_element_type=jnp.float32,
    )


def kernel(x, wg_weight):
    tokens, model_dim = x.shape
    num_experts = wg_weight.shape[0]
    return pl.pallas_call(
        _gate_kernel,
        grid=(tokens // _BT,),
        in_specs=[
            pl.BlockSpec((_BT, model_dim), lambda i: (i, 0)),
            pl.BlockSpec((num_experts, model_dim), lambda i: (0, 0)),
        ],
        out_specs=pl.BlockSpec((_BT, num_experts), lambda i: (i, 0)),
        out_shape=jax.ShapeDtypeStruct((tokens, num_experts), jnp.float32),
    )(x, wg_weight)


# speedup vs baseline: 1.0204x; 1.0204x over previous
import jax
import jax.numpy as jnp
from jax.experimental import pallas as pl

_BT = 512


def _gate_kernel(x_ref, w_ref, o_ref):
    o_ref[...] = jax.lax.dot_general(
        x_ref[...], w_ref[...],
        dimension_numbers=(((1,), (1,)), ((), ())),
        preferred_element_type=jnp.float32,
    )


def kernel(x, wg_weight):
    tokens, model_dim = x.shape
    num_experts = wg_weight.shape[0]
    return pl.pallas_call(
        _gate_kernel,
        grid=(tokens // _BT,),
        in_specs=[
            pl.BlockSpec((_BT, model_dim), lambda i: (i, 0)),
            pl.BlockSpec((num_experts, model_dim), lambda i: (0, 0)),
        ],
        out_specs=pl.BlockSpec((_BT, num_experts), lambda i: (i, 0)),
        out_shape=jax.ShapeDtypeStruct((tokens, num_experts), jnp.float32),
    )(x, wg_weight)
